# baseline (speedup 1.0000x reference)
import jax
import jax.numpy as jnp
from jax.experimental import pallas as pl
from jax.experimental.pallas import tpu as pltpu

_NMS_THRESHOLD = 0.5
_TOP_K = 400
_CONFIDENCE_THRESHOLD = 0.5
_KEEP_TOP_K = 200
_VAR0, _VAR1 = 0.1, 0.2
_NEG = -1e9
_NEGF = -3.0e38
_N_PAD = 5120
_CHUNK = 512


def _detect_body(pred_ref, pri_ref, ox1_ref, oy1_ref, ox2_ref, oy2_ref,
                 olab_ref, osc_ref,
                 tx1_ref, ty1_ref, tx2_ref, ty2_ref, tlab_ref, tsc_ref):
    B = pred_ref.shape[1]
    N = pred_ref.shape[2]

    l0 = pred_ref[0]
    l1 = pred_ref[1]
    l2 = pred_ref[2]
    l3 = pred_ref[3]
    pcx = pri_ref[0:1, :]
    pcy = pri_ref[1:2, :]
    pw = pri_ref[2:3, :]
    ph = pri_ref[3:4, :]
    cx = pcx + l0 * _VAR0 * pw
    cy = pcy + l1 * _VAR0 * ph
    w = pw * jnp.exp(l2 * _VAR1)
    h = ph * jnp.exp(l3 * _VAR1)
    x1 = cx - w / 2.0
    y1 = cy - h / 2.0
    x2 = cx + w / 2.0
    y2 = cy + h / 2.0

    m = pred_ref[4]
    lab = jnp.zeros((B, N), jnp.float32)
    for c in range(1, 21):
        cc = pred_ref[4 + c]
        gt = cc > m
        m = jnp.where(gt, cc, m)
        lab = jnp.where(gt, jnp.float32(c), lab)
    masked = jnp.where(m > _CONFIDENCE_THRESHOLD, m, _NEG)

    iota_n = jax.lax.broadcasted_iota(jnp.int32, (B, N), 1)
    lane_k = jax.lax.broadcasted_iota(jnp.int32, (1, _TOP_K), 1)
    lane_o = jax.lax.broadcasted_iota(jnp.int32, (1, _KEEP_TOP_K), 1)

    def topk_body(t, carry):
        masked, spos, ssc = carry
        mx = jnp.max(masked, axis=1, keepdims=True)
        eq = masked == mx
        idx = jnp.min(jnp.where(eq, iota_n, N), axis=1, keepdims=True)
        onehot = iota_n == idx
        tm = lane_k == t
        spos = jnp.where(tm, idx, spos)
        ssc = jnp.where(tm, mx, ssc)
        masked = jnp.where(onehot, _NEGF, masked)
        return masked, spos, ssc

    carry = (masked,
             jnp.zeros((B, _TOP_K), jnp.int32),
             jnp.zeros((B, _TOP_K), jnp.float32))
    _, spos, ssc = jax.lax.fori_loop(0, _TOP_K, topk_body, carry)

    x1t = jnp.swapaxes(x1, 0, 1)
    y1t = jnp.swapaxes(y1, 0, 1)
    x2t = jnp.swapaxes(x2, 0, 1)
    y2t = jnp.swapaxes(y2, 0, 1)
    labt = jnp.swapaxes(lab, 0, 1)
    zpad = jnp.zeros((_CHUNK, 3), jnp.float32)
    for b in range(B):
        posb = jnp.swapaxes(spos[b:b + 1, :], 0, 1)
        acc = jnp.zeros((_TOP_K, 8), jnp.float32)
        for k in range(N // _CHUNK):
            lo = k * _CHUNK
            ioch = (jax.lax.broadcasted_iota(jnp.int32, (_TOP_K, _CHUNK), 1)
                    + lo)
            oh = (posb == ioch).astype(jnp.float32)
            v = jnp.concatenate(
                [x1t[lo:lo + _CHUNK, b:b + 1],
                 y1t[lo:lo + _CHUNK, b:b + 1],
                 x2t[lo:lo + _CHUNK, b:b + 1],
                 y2t[lo:lo + _CHUNK, b:b + 1],
                 labt[lo:lo + _CHUNK, b:b + 1],
                 zpad], axis=1)
            acc = acc + jnp.dot(oh, v, preferred_element_type=jnp.float32,
                                precision=jax.lax.Precision.HIGHEST)
        tx1_ref[:, b:b + 1] = acc[:, 0:1]
        ty1_ref[:, b:b + 1] = acc[:, 1:2]
        tx2_ref[:, b:b + 1] = acc[:, 2:3]
        ty2_ref[:, b:b + 1] = acc[:, 3:4]
        tlab_ref[:, b:b + 1] = acc[:, 4:5]
    tsc_ref[...] = jnp.swapaxes(ssc, 0, 1)

    sx1 = jnp.swapaxes(tx1_ref[...], 0, 1)
    sy1 = jnp.swapaxes(ty1_ref[...], 0, 1)
    sx2 = jnp.swapaxes(tx2_ref[...], 0, 1)
    sy2 = jnp.swapaxes(ty2_ref[...], 0, 1)
    area = (jnp.clip(sx2 - sx1, 0.0, None) *
            jnp.clip(sy2 - sy1, 0.0, None))

    def ext(mask, a):
        return jnp.sum(jnp.where(mask, a, 0.0), axis=1, keepdims=True)

    def row(ref, i):
        return jnp.swapaxes(ref[pl.ds(i, 1), :], 0, 1)

    def nms_body(i, keep):
        bx1 = row(tx1_ref, i)
        by1 = row(ty1_ref, i)
        bx2 = row(tx2_ref, i)
        by2 = row(ty2_ref, i)
        bsc = row(tsc_ref, i)
        bar = (jnp.clip(bx2 - bx1, 0.0, None) *
               jnp.clip(by2 - by1, 0.0, None))
        ki = (ext(lane_k == i, keep) *
              (bsc > _CONFIDENCE_THRESHOLD).astype(jnp.float32))
        ltx = jnp.maximum(bx1, sx1)
        lty = jnp.maximum(by1, sy1)
        rbx = jnp.minimum(bx2, sx2)
        rby = jnp.minimum(by2, sy2)
        iw = jnp.clip(rbx - ltx, 0.0, None)
        ih = jnp.clip(rby - lty, 0.0, None)
        inter = iw * ih
        union = bar + area - inter
        iou = inter / jnp.maximum(union, 1e-9)
        sup = ((iou > _NMS_THRESHOLD) & (lane_k > i) & (ki > 0.5))
        keep = keep * (1.0 - sup.astype(jnp.float32))
        keep = jnp.where(lane_k == i, ki, keep)
        return keep

    keep = jax.lax.fori_loop(0, _TOP_K, nms_body,
                             jnp.ones((B, _TOP_K), jnp.float32))

    def scatter_body(i, carry):
        ox1, oy1, ox2, oy2, olab, osc = carry
        yi = row(ty1_ref, i)
        kpi = ext(lane_k == i, keep)
        less = (keep > 0.5) & ((sy1 < yi) | ((sy1 == yi) & (lane_k < i)))
        rank = jnp.sum(less.astype(jnp.int32), axis=1, keepdims=True)
        wm = (lane_o == rank) & (kpi > 0.5)
        ox1 = jnp.where(wm, row(tx1_ref, i), ox1)
        oy1 = jnp.where(wm, yi, oy1)
        ox2 = jnp.where(wm, row(tx2_ref, i), ox2)
        oy2 = jnp.where(wm, row(ty2_ref, i), oy2)
        olab = jnp.where(wm, row(tlab_ref, i), olab)
        osc = jnp.where(wm, row(tsc_ref, i), osc)
        return ox1, oy1, ox2, oy2, olab, osc

    zo = jnp.zeros((B, _KEEP_TOP_K), jnp.float32)
    ox1, oy1, ox2, oy2, olab, osc = jax.lax.fori_loop(
        0, _TOP_K, scatter_body, (zo, zo, zo, zo, zo, zo))

    ox1_ref[...] = ox1
    oy1_ref[...] = oy1
    ox2_ref[...] = ox2
    oy2_ref[...] = oy2
    olab_ref[...] = olab
    osc_ref[...] = osc


@jax.jit
def kernel(predictions, priors):
    B, N, C = predictions.shape
    pred_t = jnp.transpose(predictions, (2, 0, 1))
    pred_t = jnp.pad(pred_t, ((0, 0), (0, 0), (0, _N_PAD - N)),
                     constant_values=_NEG)
    pri_t = jnp.pad(priors.T, ((0, 0), (0, _N_PAD - N)))

    outs = pl.pallas_call(
        _detect_body,
        out_shape=[jax.ShapeDtypeStruct((B, _KEEP_TOP_K), jnp.float32)
                   for _ in range(6)],
        scratch_shapes=[pltpu.VMEM((_TOP_K, B), jnp.float32)
                        for _ in range(6)],
    )(pred_t, pri_t)
    return jnp.stack(outs, axis=-1)


# speedup vs baseline: 1.2013x; 1.2013x over previous
import jax
import jax.numpy as jnp
from jax.experimental import pallas as pl
from jax.experimental.pallas import tpu as pltpu

_NMS_THRESHOLD = 0.5
_TOP_K = 400
_CONFIDENCE_THRESHOLD = 0.5
_KEEP_TOP_K = 200
_VAR0, _VAR1 = 0.1, 0.2
_NEG = -1e9
_NEGF = -3.0e38
_N_PAD = 5120
_CHUNK = 512


def _detect_body(pred_ref, pri_ref, ox1_ref, oy1_ref, ox2_ref, oy2_ref,
                 olab_ref, osc_ref,
                 tx1_ref, ty1_ref, tx2_ref, ty2_ref, tlab_ref, tsc_ref):
    B = pred_ref.shape[1]
    N = pred_ref.shape[2]

    l0 = pred_ref[0]
    l1 = pred_ref[1]
    l2 = pred_ref[2]
    l3 = pred_ref[3]
    pcx = pri_ref[0:1, :]
    pcy = pri_ref[1:2, :]
    pw = pri_ref[2:3, :]
    ph = pri_ref[3:4, :]
    cx = pcx + l0 * _VAR0 * pw
    cy = pcy + l1 * _VAR0 * ph
    w = pw * jnp.exp(l2 * _VAR1)
    h = ph * jnp.exp(l3 * _VAR1)
    x1 = cx - w / 2.0
    y1 = cy - h / 2.0
    x2 = cx + w / 2.0
    y2 = cy + h / 2.0

    m = pred_ref[4]
    lab = jnp.zeros((B, N), jnp.float32)
    for c in range(1, 21):
        cc = pred_ref[4 + c]
        gt = cc > m
        m = jnp.where(gt, cc, m)
        lab = jnp.where(gt, jnp.float32(c), lab)
    masked = jnp.where(m > _CONFIDENCE_THRESHOLD, m, _NEG)

    iota_n = jax.lax.broadcasted_iota(jnp.int32, (B, N), 1)
    lane_k = jax.lax.broadcasted_iota(jnp.int32, (1, _TOP_K), 1)
    lane_o = jax.lax.broadcasted_iota(jnp.int32, (1, _KEEP_TOP_K), 1)

    def topk_body(t, carry):
        masked, spos, ssc = carry
        mx = jnp.max(masked, axis=1, keepdims=True)
        eq = masked == mx
        idx = jnp.min(jnp.where(eq, iota_n, N), axis=1, keepdims=True)
        onehot = iota_n == idx
        tm = lane_k == t
        spos = jnp.where(tm, idx, spos)
        ssc = jnp.where(tm, mx, ssc)
        masked = jnp.where(onehot, _NEGF, masked)
        return masked, spos, ssc

    carry = (masked,
             jnp.zeros((B, _TOP_K), jnp.int32),
             jnp.zeros((B, _TOP_K), jnp.float32))
    _, spos, ssc = jax.lax.fori_loop(0, _TOP_K, topk_body, carry)

    x1t = jnp.swapaxes(x1, 0, 1)
    y1t = jnp.swapaxes(y1, 0, 1)
    x2t = jnp.swapaxes(x2, 0, 1)
    y2t = jnp.swapaxes(y2, 0, 1)
    labt = jnp.swapaxes(lab, 0, 1)
    zpad = jnp.zeros((_CHUNK, 3), jnp.float32)
    for b in range(B):
        posb = jnp.swapaxes(spos[b:b + 1, :], 0, 1)
        acc = jnp.zeros((_TOP_K, 8), jnp.float32)
        for k in range(N // _CHUNK):
            lo = k * _CHUNK
            ioch = (jax.lax.broadcasted_iota(jnp.int32, (_TOP_K, _CHUNK), 1)
                    + lo)
            oh = (posb == ioch).astype(jnp.float32)
            v = jnp.concatenate(
                [x1t[lo:lo + _CHUNK, b:b + 1],
                 y1t[lo:lo + _CHUNK, b:b + 1],
                 x2t[lo:lo + _CHUNK, b:b + 1],
                 y2t[lo:lo + _CHUNK, b:b + 1],
                 labt[lo:lo + _CHUNK, b:b + 1],
                 zpad], axis=1)
            acc = acc + jnp.dot(oh, v, preferred_element_type=jnp.float32,
                                precision=jax.lax.Precision.HIGHEST)
        tx1_ref[:, b:b + 1] = acc[:, 0:1]
        ty1_ref[:, b:b + 1] = acc[:, 1:2]
        tx2_ref[:, b:b + 1] = acc[:, 2:3]
        ty2_ref[:, b:b + 1] = acc[:, 3:4]
        tlab_ref[:, b:b + 1] = acc[:, 4:5]
    tsc_ref[...] = jnp.swapaxes(ssc, 0, 1)

    sx1 = jnp.swapaxes(tx1_ref[...], 0, 1)
    sy1 = jnp.swapaxes(ty1_ref[...], 0, 1)
    sx2 = jnp.swapaxes(tx2_ref[...], 0, 1)
    sy2 = jnp.swapaxes(ty2_ref[...], 0, 1)
    slab = jnp.swapaxes(tlab_ref[...], 0, 1)
    svalid = (ssc > _CONFIDENCE_THRESHOLD).astype(jnp.float32)
    area = (jnp.clip(sx2 - sx1, 0.0, None) *
            jnp.clip(sy2 - sy1, 0.0, None))

    def ext(mask, a):
        return jnp.sum(jnp.where(mask, a, 0.0), axis=1, keepdims=True)

    def nms_body(i, keep):
        oh = lane_k == i
        bx1 = ext(oh, sx1)
        by1 = ext(oh, sy1)
        bx2 = ext(oh, sx2)
        by2 = ext(oh, sy2)
        bar = ext(oh, area)
        ki = ext(oh, keep) * ext(oh, svalid)
        ltx = jnp.maximum(bx1, sx1)
        lty = jnp.maximum(by1, sy1)
        rbx = jnp.minimum(bx2, sx2)
        rby = jnp.minimum(by2, sy2)
        iw = jnp.clip(rbx - ltx, 0.0, None)
        ih = jnp.clip(rby - lty, 0.0, None)
        inter = iw * ih
        union = bar + area - inter
        iou = inter / jnp.maximum(union, 1e-9)
        sup = ((iou > _NMS_THRESHOLD) & (lane_k > i) & (ki > 0.5))
        keep = keep * (1.0 - sup.astype(jnp.float32))
        keep = jnp.where(lane_k == i, ki, keep)
        return keep

    keep = jax.lax.fori_loop(0, _TOP_K, nms_body,
                             jnp.ones((B, _TOP_K), jnp.float32))

    def scatter_body(i, carry):
        ox1, oy1, ox2, oy2, olab, osc = carry
        oh = lane_k == i
        yi = ext(oh, sy1)
        kpi = ext(oh, keep)
        less = (keep > 0.5) & ((sy1 < yi) | ((sy1 == yi) & (lane_k < i)))
        rank = jnp.sum(less.astype(jnp.int32), axis=1, keepdims=True)
        wm = (lane_o == rank) & (kpi > 0.5)
        ox1 = jnp.where(wm, ext(oh, sx1), ox1)
        oy1 = jnp.where(wm, yi, oy1)
        ox2 = jnp.where(wm, ext(oh, sx2), ox2)
        oy2 = jnp.where(wm, ext(oh, sy2), oy2)
        olab = jnp.where(wm, ext(oh, slab), olab)
        osc = jnp.where(wm, ext(oh, ssc), osc)
        return ox1, oy1, ox2, oy2, olab, osc

    zo = jnp.zeros((B, _KEEP_TOP_K), jnp.float32)
    ox1, oy1, ox2, oy2, olab, osc = jax.lax.fori_loop(
        0, _TOP_K, scatter_body, (zo, zo, zo, zo, zo, zo))

    ox1_ref[...] = ox1
    oy1_ref[...] = oy1
    ox2_ref[...] = ox2
    oy2_ref[...] = oy2
    olab_ref[...] = olab
    osc_ref[...] = osc


@jax.jit
def kernel(predictions, priors):
    B, N, C = predictions.shape
    pred_t = jnp.transpose(predictions, (2, 0, 1))
    pred_t = jnp.pad(pred_t, ((0, 0), (0, 0), (0, _N_PAD - N)),
                     constant_values=_NEG)
    pri_t = jnp.pad(priors.T, ((0, 0), (0, _N_PAD - N)))

    outs = pl.pallas_call(
        _detect_body,
        out_shape=[jax.ShapeDtypeStruct((B, _KEEP_TOP_K), jnp.float32)
                   for _ in range(6)],
        scratch_shapes=[pltpu.VMEM((_TOP_K, B), jnp.float32)
                        for _ in range(6)],
    )(pred_t, pri_t)
    return jnp.stack(outs, axis=-1)


# speedup vs baseline: 1.2430x; 1.0347x over previous
import jax
import jax.numpy as jnp
from jax.experimental import pallas as pl
from jax.experimental.pallas import tpu as pltpu

_NMS_THRESHOLD = 0.5
_TOP_K = 400
_CONFIDENCE_THRESHOLD = 0.5
_KEEP_TOP_K = 200
_VAR0, _VAR1 = 0.1, 0.2
_NEG = -1e9
_NEGF = -3.0e38
_N_PAD = 5120
_CHUNK = 512


def _detect_body(pred_ref, pri_ref, ox1_ref, oy1_ref, ox2_ref, oy2_ref,
                 olab_ref, osc_ref,
                 tx1_ref, ty1_ref, tx2_ref, ty2_ref, tlab_ref, tsc_ref):
    B = pred_ref.shape[1]
    N = pred_ref.shape[2]

    l0 = pred_ref[0]
    l1 = pred_ref[1]
    l2 = pred_ref[2]
    l3 = pred_ref[3]
    pcx = pri_ref[0:1, :]
    pcy = pri_ref[1:2, :]
    pw = pri_ref[2:3, :]
    ph = pri_ref[3:4, :]
    cx = pcx + l0 * _VAR0 * pw
    cy = pcy + l1 * _VAR0 * ph
    w = pw * jnp.exp(l2 * _VAR1)
    h = ph * jnp.exp(l3 * _VAR1)
    x1 = cx - w / 2.0
    y1 = cy - h / 2.0
    x2 = cx + w / 2.0
    y2 = cy + h / 2.0

    m = pred_ref[4]
    lab = jnp.zeros((B, N), jnp.float32)
    for c in range(1, 21):
        cc = pred_ref[4 + c]
        gt = cc > m
        m = jnp.where(gt, cc, m)
        lab = jnp.where(gt, jnp.float32(c), lab)
    masked = jnp.where(m > _CONFIDENCE_THRESHOLD, m, _NEG)

    iota_n = jax.lax.broadcasted_iota(jnp.int32, (B, N), 1)
    lane_k = jax.lax.broadcasted_iota(jnp.int32, (1, _TOP_K), 1)
    lane_o = jax.lax.broadcasted_iota(jnp.int32, (1, _KEEP_TOP_K), 1)

    u = jax.lax.bitcast_convert_type(masked, jnp.uint32)
    neg = u >> 31
    key = u ^ ((jnp.uint32(0) - neg) | jnp.uint32(0x80000000))

    thr = jnp.zeros((B, 1), jnp.uint32)
    for b in range(31, -1, -1):
        cand = thr | jnp.uint32(1 << b)
        cnt = jnp.sum((key >= cand).astype(jnp.int32), axis=1, keepdims=True)
        thr = jnp.where(cnt >= _TOP_K, cand, thr)
    tie = key == thr
    n_gt = jnp.sum((key > thr).astype(jnp.int32), axis=1, keepdims=True)
    need = _TOP_K - n_gt
    rev = (N - 1) - iota_n
    thr_r = jnp.zeros((B, 1), jnp.int32)
    for b in range(12, -1, -1):
        cand = thr_r | (1 << b)
        cnt = jnp.sum((tie & (rev >= cand)).astype(jnp.int32),
                      axis=1, keepdims=True)
        thr_r = jnp.where(cnt >= need, cand, thr_r)
    candmask = (key > thr) | (tie & (rev >= thr_r))

    cm = candmask.astype(jnp.int32)
    incl = cm
    for b in range(13):
        d = 1 << b
        incl = incl + jnp.concatenate(
            [jnp.zeros((B, d), jnp.int32), incl[:, :N - d]], axis=1)
    shift = iota_n - (incl - cm)

    cs = masked
    cidx = iota_n
    for b in range(13):
        d = 1 << b

        def rolled(a):
            return jnp.concatenate([a[:, d:], a[:, :d]], axis=1)

        take = ((rolled(shift) >> b) & 1) == 1
        cs = jnp.where(take, rolled(cs), cs)
        cidx = jnp.where(take, rolled(cidx), cidx)
        shift = jnp.where(take, rolled(shift) - d, shift)

    cs = cs[:, :_TOP_K]
    cidx = cidx[:, :_TOP_K]

    def sort_body(t, carry):
        cs, spos, ssc = carry
        mx = jnp.max(cs, axis=1, keepdims=True)
        eq = cs == mx
        mi = jnp.min(jnp.where(eq, cidx, N), axis=1, keepdims=True)
        onehot = cidx == mi
        tm = lane_k == t
        spos = jnp.where(tm, mi, spos)
        ssc = jnp.where(tm, mx, ssc)
        cs = jnp.where(onehot, _NEGF, cs)
        return cs, spos, ssc

    carry = (cs,
             jnp.zeros((B, _TOP_K), jnp.int32),
             jnp.zeros((B, _TOP_K), jnp.float32))
    _, spos, ssc = jax.lax.fori_loop(0, _TOP_K, sort_body, carry)

    x1t = jnp.swapaxes(x1, 0, 1)
    y1t = jnp.swapaxes(y1, 0, 1)
    x2t = jnp.swapaxes(x2, 0, 1)
    y2t = jnp.swapaxes(y2, 0, 1)
    labt = jnp.swapaxes(lab, 0, 1)
    zpad = jnp.zeros((_CHUNK, 3), jnp.float32)
    for b in range(B):
        posb = jnp.swapaxes(spos[b:b + 1, :], 0, 1)
        acc = jnp.zeros((_TOP_K, 8), jnp.float32)
        for k in range(N // _CHUNK):
            lo = k * _CHUNK
            ioch = (jax.lax.broadcasted_iota(jnp.int32, (_TOP_K, _CHUNK), 1)
                    + lo)
            oh = (posb == ioch).astype(jnp.float32)
            v = jnp.concatenate(
                [x1t[lo:lo + _CHUNK, b:b + 1],
                 y1t[lo:lo + _CHUNK, b:b + 1],
                 x2t[lo:lo + _CHUNK, b:b + 1],
                 y2t[lo:lo + _CHUNK, b:b + 1],
                 labt[lo:lo + _CHUNK, b:b + 1],
                 zpad], axis=1)
            acc = acc + jnp.dot(oh, v, preferred_element_type=jnp.float32,
                                precision=jax.lax.Precision.HIGHEST)
        tx1_ref[:, b:b + 1] = acc[:, 0:1]
        ty1_ref[:, b:b + 1] = acc[:, 1:2]
        tx2_ref[:, b:b + 1] = acc[:, 2:3]
        ty2_ref[:, b:b + 1] = acc[:, 3:4]
        tlab_ref[:, b:b + 1] = acc[:, 4:5]
    tsc_ref[...] = jnp.swapaxes(ssc, 0, 1)

    sx1 = jnp.swapaxes(tx1_ref[...], 0, 1)
    sy1 = jnp.swapaxes(ty1_ref[...], 0, 1)
    sx2 = jnp.swapaxes(tx2_ref[...], 0, 1)
    sy2 = jnp.swapaxes(ty2_ref[...], 0, 1)
    slab = jnp.swapaxes(tlab_ref[...], 0, 1)
    svalid = (ssc > _CONFIDENCE_THRESHOLD).astype(jnp.float32)
    area = (jnp.clip(sx2 - sx1, 0.0, None) *
            jnp.clip(sy2 - sy1, 0.0, None))

    def ext(mask, a):
        return jnp.sum(jnp.where(mask, a, 0.0), axis=1, keepdims=True)

    def nms_body(i, keep):
        oh = lane_k == i
        bx1 = ext(oh, sx1)
        by1 = ext(oh, sy1)
        bx2 = ext(oh, sx2)
        by2 = ext(oh, sy2)
        bar = ext(oh, area)
        ki = ext(oh, keep) * ext(oh, svalid)
        ltx = jnp.maximum(bx1, sx1)
        lty = jnp.maximum(by1, sy1)
        rbx = jnp.minimum(bx2, sx2)
        rby = jnp.minimum(by2, sy2)
        iw = jnp.clip(rbx - ltx, 0.0, None)
        ih = jnp.clip(rby - lty, 0.0, None)
        inter = iw * ih
        union = bar + area - inter
        iou = inter / jnp.maximum(union, 1e-9)
        sup = ((iou > _NMS_THRESHOLD) & (lane_k > i) & (ki > 0.5))
        keep = keep * (1.0 - sup.astype(jnp.float32))
        keep = jnp.where(lane_k == i, ki, keep)
        return keep

    keep = jax.lax.fori_loop(0, _TOP_K, nms_body,
                             jnp.ones((B, _TOP_K), jnp.float32))

    def scatter_body(i, carry):
        ox1, oy1, ox2, oy2, olab, osc = carry
        oh = lane_k == i
        yi = ext(oh, sy1)
        kpi = ext(oh, keep)
        less = (keep > 0.5) & ((sy1 < yi) | ((sy1 == yi) & (lane_k < i)))
        rank = jnp.sum(less.astype(jnp.int32), axis=1, keepdims=True)
        wm = (lane_o == rank) & (kpi > 0.5)
        ox1 = jnp.where(wm, ext(oh, sx1), ox1)
        oy1 = jnp.where(wm, yi, oy1)
        ox2 = jnp.where(wm, ext(oh, sx2), ox2)
        oy2 = jnp.where(wm, ext(oh, sy2), oy2)
        olab = jnp.where(wm, ext(oh, slab), olab)
        osc = jnp.where(wm, ext(oh, ssc), osc)
        return ox1, oy1, ox2, oy2, olab, osc

    zo = jnp.zeros((B, _KEEP_TOP_K), jnp.float32)
    ox1, oy1, ox2, oy2, olab, osc = jax.lax.fori_loop(
        0, _TOP_K, scatter_body, (zo, zo, zo, zo, zo, zo))

    ox1_ref[...] = ox1
    oy1_ref[...] = oy1
    ox2_ref[...] = ox2
    oy2_ref[...] = oy2
    olab_ref[...] = olab
    osc_ref[...] = osc


@jax.jit
def kernel(predictions, priors):
    B, N, C = predictions.shape
    pred_t = jnp.transpose(predictions, (2, 0, 1))
    pred_t = jnp.pad(pred_t, ((0, 0), (0, 0), (0, _N_PAD - N)),
                     constant_values=_NEG)
    pri_t = jnp.pad(priors.T, ((0, 0), (0, _N_PAD - N)))

    outs = pl.pallas_call(
        _detect_body,
        out_shape=[jax.ShapeDtypeStruct((B, _KEEP_TOP_K), jnp.float32)
                   for _ in range(6)],
        scratch_shapes=[pltpu.VMEM((_TOP_K, B), jnp.float32)
                        for _ in range(6)],
    )(pred_t, pri_t)
    return jnp.stack(outs, axis=-1)


# speedup vs baseline: 1.5496x; 1.2467x over previous
import jax
import jax.numpy as jnp
from jax.experimental import pallas as pl
from jax.experimental.pallas import tpu as pltpu

_NMS_THRESHOLD = 0.5
_TOP_K = 400
_CONFIDENCE_THRESHOLD = 0.5
_KEEP_TOP_K = 200
_VAR0, _VAR1 = 0.1, 0.2
_NEG = -1e9
_NEGF = -3.0e38
_N_PAD = 5120
_CHUNK = 512


def _detect_body(pred_ref, pri_ref, ox1_ref, oy1_ref, ox2_ref, oy2_ref,
                 olab_ref, osc_ref,
                 tx1_ref, ty1_ref, tx2_ref, ty2_ref, tlab_ref, tsc_ref):
    B = pred_ref.shape[1]
    N = pred_ref.shape[2]

    l0 = pred_ref[0]
    l1 = pred_ref[1]
    l2 = pred_ref[2]
    l3 = pred_ref[3]
    pcx = pri_ref[0:1, :]
    pcy = pri_ref[1:2, :]
    pw = pri_ref[2:3, :]
    ph = pri_ref[3:4, :]
    cx = pcx + l0 * _VAR0 * pw
    cy = pcy + l1 * _VAR0 * ph
    w = pw * jnp.exp(l2 * _VAR1)
    h = ph * jnp.exp(l3 * _VAR1)
    x1 = cx - w / 2.0
    y1 = cy - h / 2.0
    x2 = cx + w / 2.0
    y2 = cy + h / 2.0

    m = pred_ref[4]
    lab = jnp.zeros((B, N), jnp.float32)
    for c in range(1, 21):
        cc = pred_ref[4 + c]
        gt = cc > m
        m = jnp.where(gt, cc, m)
        lab = jnp.where(gt, jnp.float32(c), lab)
    masked = jnp.where(m > _CONFIDENCE_THRESHOLD, m, _NEG)

    iota_n = jax.lax.broadcasted_iota(jnp.int32, (B, N), 1)
    lane_k = jax.lax.broadcasted_iota(jnp.int32, (1, _TOP_K), 1)
    lane_o = jax.lax.broadcasted_iota(jnp.int32, (1, _KEEP_TOP_K), 1)

    u = jax.lax.bitcast_convert_type(masked, jnp.uint32)
    neg = u >> 31
    key = u ^ ((jnp.uint32(0) - neg) | jnp.uint32(0x80000000))

    thr = jnp.zeros((B, 1), jnp.uint32)
    for b in range(31, -1, -1):
        cand = thr | jnp.uint32(1 << b)
        cnt = jnp.sum((key >= cand).astype(jnp.int32), axis=1, keepdims=True)
        thr = jnp.where(cnt >= _TOP_K, cand, thr)
    tie = key == thr
    n_gt = jnp.sum((key > thr).astype(jnp.int32), axis=1, keepdims=True)
    need = _TOP_K - n_gt
    rev = (N - 1) - iota_n
    thr_r = jnp.zeros((B, 1), jnp.int32)
    for b in range(12, -1, -1):
        cand = thr_r | (1 << b)
        cnt = jnp.sum((tie & (rev >= cand)).astype(jnp.int32),
                      axis=1, keepdims=True)
        thr_r = jnp.where(cnt >= need, cand, thr_r)
    candmask = (key > thr) | (tie & (rev >= thr_r))

    cm = candmask.astype(jnp.int32)
    incl = cm
    for b in range(13):
        d = 1 << b
        incl = incl + jnp.concatenate(
            [jnp.zeros((B, d), jnp.int32), incl[:, :N - d]], axis=1)
    shift = iota_n - (incl - cm)

    cs = masked
    cidx = iota_n
    for b in range(13):
        d = 1 << b

        def rolled(a):
            return jnp.concatenate([a[:, d:], a[:, :d]], axis=1)

        take = ((rolled(shift) >> b) & 1) == 1
        cs = jnp.where(take, rolled(cs), cs)
        cidx = jnp.where(take, rolled(cidx), cidx)
        shift = jnp.where(take, rolled(shift) - d, shift)

    cs = cs[:, :_TOP_K]
    cidx = cidx[:, :_TOP_K]

    def sort_body(t, carry):
        cs, spos, ssc = carry
        mx = jnp.max(cs, axis=1, keepdims=True)
        eq = cs == mx
        mi = jnp.min(jnp.where(eq, cidx, N), axis=1, keepdims=True)
        onehot = cidx == mi
        tm = lane_k == t
        spos = jnp.where(tm, mi, spos)
        ssc = jnp.where(tm, mx, ssc)
        cs = jnp.where(onehot, _NEGF, cs)
        return cs, spos, ssc

    carry = (cs,
             jnp.zeros((B, _TOP_K), jnp.int32),
             jnp.zeros((B, _TOP_K), jnp.float32))
    def sort_body4(t4, carry):
        for s in range(4):
            carry = sort_body(t4 * 4 + s, carry)
        return carry

    _, spos, ssc = jax.lax.fori_loop(0, _TOP_K // 4, sort_body4, carry)

    x1t = jnp.swapaxes(x1, 0, 1)
    y1t = jnp.swapaxes(y1, 0, 1)
    x2t = jnp.swapaxes(x2, 0, 1)
    y2t = jnp.swapaxes(y2, 0, 1)
    labt = jnp.swapaxes(lab, 0, 1)
    zpad = jnp.zeros((_CHUNK, 3), jnp.float32)
    for b in range(B):
        posb = jnp.swapaxes(spos[b:b + 1, :], 0, 1)
        acc = jnp.zeros((_TOP_K, 8), jnp.float32)
        for k in range(N // _CHUNK):
            lo = k * _CHUNK
            ioch = (jax.lax.broadcasted_iota(jnp.int32, (_TOP_K, _CHUNK), 1)
                    + lo)
            oh = (posb == ioch).astype(jnp.float32)
            v = jnp.concatenate(
                [x1t[lo:lo + _CHUNK, b:b + 1],
                 y1t[lo:lo + _CHUNK, b:b + 1],
                 x2t[lo:lo + _CHUNK, b:b + 1],
                 y2t[lo:lo + _CHUNK, b:b + 1],
                 labt[lo:lo + _CHUNK, b:b + 1],
                 zpad], axis=1)
            acc = acc + jnp.dot(oh, v, preferred_element_type=jnp.float32,
                                precision=jax.lax.Precision.HIGHEST)
        tx1_ref[:, b:b + 1] = acc[:, 0:1]
        ty1_ref[:, b:b + 1] = acc[:, 1:2]
        tx2_ref[:, b:b + 1] = acc[:, 2:3]
        ty2_ref[:, b:b + 1] = acc[:, 3:4]
        tlab_ref[:, b:b + 1] = acc[:, 4:5]
    tsc_ref[...] = jnp.swapaxes(ssc, 0, 1)

    sx1 = jnp.swapaxes(tx1_ref[...], 0, 1)
    sy1 = jnp.swapaxes(ty1_ref[...], 0, 1)
    sx2 = jnp.swapaxes(tx2_ref[...], 0, 1)
    sy2 = jnp.swapaxes(ty2_ref[...], 0, 1)
    slab = jnp.swapaxes(tlab_ref[...], 0, 1)
    svalid = (ssc > _CONFIDENCE_THRESHOLD).astype(jnp.float32)
    area = (jnp.clip(sx2 - sx1, 0.0, None) *
            jnp.clip(sy2 - sy1, 0.0, None))

    def ext(mask, a):
        return jnp.sum(jnp.where(mask, a, 0.0), axis=1, keepdims=True)

    def nms_body(i, keep):
        oh = lane_k == i
        bx1 = ext(oh, sx1)
        by1 = ext(oh, sy1)
        bx2 = ext(oh, sx2)
        by2 = ext(oh, sy2)
        bar = ext(oh, area)
        ki = ext(oh, keep) * ext(oh, svalid)
        ltx = jnp.maximum(bx1, sx1)
        lty = jnp.maximum(by1, sy1)
        rbx = jnp.minimum(bx2, sx2)
        rby = jnp.minimum(by2, sy2)
        iw = jnp.clip(rbx - ltx, 0.0, None)
        ih = jnp.clip(rby - lty, 0.0, None)
        inter = iw * ih
        union = bar + area - inter
        iou = inter / jnp.maximum(union, 1e-9)
        sup = ((iou > _NMS_THRESHOLD) & (lane_k > i) & (ki > 0.5))
        keep = keep * (1.0 - sup.astype(jnp.float32))
        keep = jnp.where(lane_k == i, ki, keep)
        return keep

    def nms_body4(i4, keep):
        for s in range(4):
            keep = nms_body(i4 * 4 + s, keep)
        return keep

    keep = jax.lax.fori_loop(0, _TOP_K // 4, nms_body4,
                             jnp.ones((B, _TOP_K), jnp.float32))

    def scatter_body(i, carry):
        ox1, oy1, ox2, oy2, olab, osc = carry
        oh = lane_k == i
        yi = ext(oh, sy1)
        kpi = ext(oh, keep)
        less = (keep > 0.5) & ((sy1 < yi) | ((sy1 == yi) & (lane_k < i)))
        rank = jnp.sum(less.astype(jnp.int32), axis=1, keepdims=True)
        wm = (lane_o == rank) & (kpi > 0.5)
        ox1 = jnp.where(wm, ext(oh, sx1), ox1)
        oy1 = jnp.where(wm, yi, oy1)
        ox2 = jnp.where(wm, ext(oh, sx2), ox2)
        oy2 = jnp.where(wm, ext(oh, sy2), oy2)
        olab = jnp.where(wm, ext(oh, slab), olab)
        osc = jnp.where(wm, ext(oh, ssc), osc)
        return ox1, oy1, ox2, oy2, olab, osc

    zo = jnp.zeros((B, _KEEP_TOP_K), jnp.float32)
    def scatter_body4(i4, carry):
        for s in range(4):
            carry = scatter_body(i4 * 4 + s, carry)
        return carry

    ox1, oy1, ox2, oy2, olab, osc = jax.lax.fori_loop(
        0, _TOP_K // 4, scatter_body4, (zo, zo, zo, zo, zo, zo))

    ox1_ref[...] = ox1
    oy1_ref[...] = oy1
    ox2_ref[...] = ox2
    oy2_ref[...] = oy2
    olab_ref[...] = olab
    osc_ref[...] = osc


@jax.jit
def kernel(predictions, priors):
    B, N, C = predictions.shape
    pred_t = jnp.transpose(predictions, (2, 0, 1))
    pred_t = jnp.pad(pred_t, ((0, 0), (0, 0), (0, _N_PAD - N)),
                     constant_values=_NEG)
    pri_t = jnp.pad(priors.T, ((0, 0), (0, _N_PAD - N)))

    outs = pl.pallas_call(
        _detect_body,
        out_shape=[jax.ShapeDtypeStruct((B, _KEEP_TOP_K), jnp.float32)
                   for _ in range(6)],
        scratch_shapes=[pltpu.VMEM((_TOP_K, B), jnp.float32)
                        for _ in range(6)],
    )(pred_t, pri_t)
    return jnp.stack(outs, axis=-1)


# speedup vs baseline: 1.6196x; 1.0452x over previous
import jax
import jax.numpy as jnp
from jax.experimental import pallas as pl
from jax.experimental.pallas import tpu as pltpu

_NMS_THRESHOLD = 0.5
_TOP_K = 400
_CONFIDENCE_THRESHOLD = 0.5
_KEEP_TOP_K = 200
_VAR0, _VAR1 = 0.1, 0.2
_NEG = -1e9
_NEGF = -3.0e38
_N_PAD = 5120
_CHUNK = 512


def _detect_body(pred_ref, pri_ref, ox1_ref, oy1_ref, ox2_ref, oy2_ref,
                 olab_ref, osc_ref,
                 tx1_ref, ty1_ref, tx2_ref, ty2_ref, tlab_ref, tsc_ref):
    B = pred_ref.shape[1]
    N = pred_ref.shape[2]

    l0 = pred_ref[0]
    l1 = pred_ref[1]
    l2 = pred_ref[2]
    l3 = pred_ref[3]
    pcx = pri_ref[0:1, :]
    pcy = pri_ref[1:2, :]
    pw = pri_ref[2:3, :]
    ph = pri_ref[3:4, :]
    cx = pcx + l0 * _VAR0 * pw
    cy = pcy + l1 * _VAR0 * ph
    w = pw * jnp.exp(l2 * _VAR1)
    h = ph * jnp.exp(l3 * _VAR1)
    x1 = cx - w / 2.0
    y1 = cy - h / 2.0
    x2 = cx + w / 2.0
    y2 = cy + h / 2.0

    m = pred_ref[4]
    lab = jnp.zeros((B, N), jnp.float32)
    for c in range(1, 21):
        cc = pred_ref[4 + c]
        gt = cc > m
        m = jnp.where(gt, cc, m)
        lab = jnp.where(gt, jnp.float32(c), lab)
    masked = jnp.where(m > _CONFIDENCE_THRESHOLD, m, _NEG)

    iota_n = jax.lax.broadcasted_iota(jnp.int32, (B, N), 1)
    lane_k = jax.lax.broadcasted_iota(jnp.int32, (1, _TOP_K), 1)
    lane_o = jax.lax.broadcasted_iota(jnp.int32, (1, _KEEP_TOP_K), 1)

    u = jax.lax.bitcast_convert_type(masked, jnp.uint32)
    neg = u >> 31
    key = u ^ ((jnp.uint32(0) - neg) | jnp.uint32(0x80000000))

    thr = jnp.zeros((B, 1), jnp.uint32)
    for b in range(31, -1, -1):
        cand = thr | jnp.uint32(1 << b)
        cnt = jnp.sum((key >= cand).astype(jnp.int32), axis=1, keepdims=True)
        thr = jnp.where(cnt >= _TOP_K, cand, thr)
    tie = key == thr
    n_gt = jnp.sum((key > thr).astype(jnp.int32), axis=1, keepdims=True)
    need = _TOP_K - n_gt
    rev = (N - 1) - iota_n
    thr_r = jnp.zeros((B, 1), jnp.int32)
    for b in range(12, -1, -1):
        cand = thr_r | (1 << b)
        cnt = jnp.sum((tie & (rev >= cand)).astype(jnp.int32),
                      axis=1, keepdims=True)
        thr_r = jnp.where(cnt >= need, cand, thr_r)
    candmask = (key > thr) | (tie & (rev >= thr_r))

    cm = candmask.astype(jnp.int32)
    incl = cm
    for b in range(13):
        d = 1 << b
        incl = incl + jnp.concatenate(
            [jnp.zeros((B, d), jnp.int32), incl[:, :N - d]], axis=1)
    shift = iota_n - (incl - cm)

    cs = masked
    cidx = iota_n
    for b in range(13):
        d = 1 << b

        def rolled(a):
            return jnp.concatenate([a[:, d:], a[:, :d]], axis=1)

        take = ((rolled(shift) >> b) & 1) == 1
        cs = jnp.where(take, rolled(cs), cs)
        cidx = jnp.where(take, rolled(cidx), cidx)
        shift = jnp.where(take, rolled(shift) - d, shift)

    cs = cs[:, :_TOP_K]
    cidx = cidx[:, :_TOP_K]

    def sort_body(t, carry):
        cs, spos, ssc = carry
        mx = jnp.max(cs, axis=1, keepdims=True)
        eq = cs == mx
        mi = jnp.min(jnp.where(eq, cidx, N), axis=1, keepdims=True)
        onehot = cidx == mi
        tm = lane_k == t
        spos = jnp.where(tm, mi, spos)
        ssc = jnp.where(tm, mx, ssc)
        cs = jnp.where(onehot, _NEGF, cs)
        return cs, spos, ssc

    carry = (cs,
             jnp.zeros((B, _TOP_K), jnp.int32),
             jnp.zeros((B, _TOP_K), jnp.float32))
    def sort_body4(t4, carry):
        for s in range(8):
            carry = sort_body(t4 * 8 + s, carry)
        return carry

    _, spos, ssc = jax.lax.fori_loop(0, _TOP_K // 8, sort_body4, carry)

    x1t = jnp.swapaxes(x1, 0, 1)
    y1t = jnp.swapaxes(y1, 0, 1)
    x2t = jnp.swapaxes(x2, 0, 1)
    y2t = jnp.swapaxes(y2, 0, 1)
    labt = jnp.swapaxes(lab, 0, 1)
    zpad = jnp.zeros((_CHUNK, 3), jnp.float32)
    for b in range(B):
        posb = jnp.swapaxes(spos[b:b + 1, :], 0, 1)
        acc = jnp.zeros((_TOP_K, 8), jnp.float32)
        for k in range(N // _CHUNK):
            lo = k * _CHUNK
            ioch = (jax.lax.broadcasted_iota(jnp.int32, (_TOP_K, _CHUNK), 1)
                    + lo)
            oh = (posb == ioch).astype(jnp.float32)
            v = jnp.concatenate(
                [x1t[lo:lo + _CHUNK, b:b + 1],
                 y1t[lo:lo + _CHUNK, b:b + 1],
                 x2t[lo:lo + _CHUNK, b:b + 1],
                 y2t[lo:lo + _CHUNK, b:b + 1],
                 labt[lo:lo + _CHUNK, b:b + 1],
                 zpad], axis=1)
            acc = acc + jnp.dot(oh, v, preferred_element_type=jnp.float32,
                                precision=jax.lax.Precision.HIGHEST)
        tx1_ref[:, b:b + 1] = acc[:, 0:1]
        ty1_ref[:, b:b + 1] = acc[:, 1:2]
        tx2_ref[:, b:b + 1] = acc[:, 2:3]
        ty2_ref[:, b:b + 1] = acc[:, 3:4]
        tlab_ref[:, b:b + 1] = acc[:, 4:5]
    tsc_ref[...] = jnp.swapaxes(ssc, 0, 1)

    sx1 = jnp.swapaxes(tx1_ref[...], 0, 1)
    sy1 = jnp.swapaxes(ty1_ref[...], 0, 1)
    sx2 = jnp.swapaxes(tx2_ref[...], 0, 1)
    sy2 = jnp.swapaxes(ty2_ref[...], 0, 1)
    slab = jnp.swapaxes(tlab_ref[...], 0, 1)
    svalid = (ssc > _CONFIDENCE_THRESHOLD).astype(jnp.float32)
    area = (jnp.clip(sx2 - sx1, 0.0, None) *
            jnp.clip(sy2 - sy1, 0.0, None))

    def ext(mask, a):
        return jnp.sum(jnp.where(mask, a, 0.0), axis=1, keepdims=True)

    def nms_body(i, keep):
        oh = lane_k == i
        bx1 = ext(oh, sx1)
        by1 = ext(oh, sy1)
        bx2 = ext(oh, sx2)
        by2 = ext(oh, sy2)
        bar = ext(oh, area)
        ki = ext(oh, keep) * ext(oh, svalid)
        ltx = jnp.maximum(bx1, sx1)
        lty = jnp.maximum(by1, sy1)
        rbx = jnp.minimum(bx2, sx2)
        rby = jnp.minimum(by2, sy2)
        iw = jnp.clip(rbx - ltx, 0.0, None)
        ih = jnp.clip(rby - lty, 0.0, None)
        inter = iw * ih
        union = bar + area - inter
        iou = inter / jnp.maximum(union, 1e-9)
        sup = ((iou > _NMS_THRESHOLD) & (lane_k > i) & (ki > 0.5))
        keep = keep * (1.0 - sup.astype(jnp.float32))
        keep = jnp.where(lane_k == i, ki, keep)
        return keep

    def nms_body4(i4, keep):
        for s in range(8):
            keep = nms_body(i4 * 8 + s, keep)
        return keep

    keep = jax.lax.fori_loop(0, _TOP_K // 8, nms_body4,
                             jnp.ones((B, _TOP_K), jnp.float32))

    def scatter_body(i, carry):
        ox1, oy1, ox2, oy2, olab, osc = carry
        oh = lane_k == i
        yi = ext(oh, sy1)
        kpi = ext(oh, keep)
        less = (keep > 0.5) & ((sy1 < yi) | ((sy1 == yi) & (lane_k < i)))
        rank = jnp.sum(less.astype(jnp.int32), axis=1, keepdims=True)
        wm = (lane_o == rank) & (kpi > 0.5)
        ox1 = jnp.where(wm, ext(oh, sx1), ox1)
        oy1 = jnp.where(wm, yi, oy1)
        ox2 = jnp.where(wm, ext(oh, sx2), ox2)
        oy2 = jnp.where(wm, ext(oh, sy2), oy2)
        olab = jnp.where(wm, ext(oh, slab), olab)
        osc = jnp.where(wm, ext(oh, ssc), osc)
        return ox1, oy1, ox2, oy2, olab, osc

    zo = jnp.zeros((B, _KEEP_TOP_K), jnp.float32)
    def scatter_body4(i4, carry):
        for s in range(8):
            carry = scatter_body(i4 * 8 + s, carry)
        return carry

    ox1, oy1, ox2, oy2, olab, osc = jax.lax.fori_loop(
        0, _TOP_K // 8, scatter_body4, (zo, zo, zo, zo, zo, zo))

    ox1_ref[...] = ox1
    oy1_ref[...] = oy1
    ox2_ref[...] = ox2
    oy2_ref[...] = oy2
    olab_ref[...] = olab
    osc_ref[...] = osc


@jax.jit
def kernel(predictions, priors):
    B, N, C = predictions.shape
    pred_t = jnp.transpose(predictions, (2, 0, 1))
    pred_t = jnp.pad(pred_t, ((0, 0), (0, 0), (0, _N_PAD - N)),
                     constant_values=_NEG)
    pri_t = jnp.pad(priors.T, ((0, 0), (0, _N_PAD - N)))

    outs = pl.pallas_call(
        _detect_body,
        out_shape=[jax.ShapeDtypeStruct((B, _KEEP_TOP_K), jnp.float32)
                   for _ in range(6)],
        scratch_shapes=[pltpu.VMEM((_TOP_K, B), jnp.float32)
                        for _ in range(6)],
    )(pred_t, pri_t)
    return jnp.stack(outs, axis=-1)


# speedup vs baseline: 2.1182x; 1.3079x over previous
import jax
import jax.numpy as jnp
from jax.experimental import pallas as pl

_NMS_THRESHOLD = 0.5
_TOP_K = 400
_CONFIDENCE_THRESHOLD = 0.5
_KEEP_TOP_K = 200
_VAR0, _VAR1 = 0.1, 0.2
_NEG = -1e9
_NEGF = -3.0e38
_N_PAD = 5120
_UNROLL = 8


def _detect_body(pred_ref, pri_ref, ox1_ref, oy1_ref, ox2_ref, oy2_ref,
                 olab_ref, osc_ref):
    B = pred_ref.shape[1]
    N = pred_ref.shape[2]

    l0 = pred_ref[0]
    l1 = pred_ref[1]
    l2 = pred_ref[2]
    l3 = pred_ref[3]
    pcx = pri_ref[0:1, :]
    pcy = pri_ref[1:2, :]
    pw = pri_ref[2:3, :]
    ph = pri_ref[3:4, :]
    cx = pcx + l0 * _VAR0 * pw
    cy = pcy + l1 * _VAR0 * ph
    w = pw * jnp.exp(l2 * _VAR1)
    h = ph * jnp.exp(l3 * _VAR1)
    x1 = cx - w / 2.0
    y1 = cy - h / 2.0
    x2 = cx + w / 2.0
    y2 = cy + h / 2.0

    m = pred_ref[4]
    lab = jnp.zeros((B, N), jnp.float32)
    for c in range(1, 21):
        cc = pred_ref[4 + c]
        gt = cc > m
        m = jnp.where(gt, cc, m)
        lab = jnp.where(gt, jnp.float32(c), lab)
    masked = jnp.where(m > _CONFIDENCE_THRESHOLD, m, _NEG)

    iota_n = jax.lax.broadcasted_iota(jnp.int32, (B, N), 1)
    lane_k = jax.lax.broadcasted_iota(jnp.int32, (1, _TOP_K), 1)

    u = jax.lax.bitcast_convert_type(masked, jnp.uint32)
    neg = u >> 31
    key = u ^ ((jnp.uint32(0) - neg) | jnp.uint32(0x80000000))

    thr = jnp.zeros((B, 1), jnp.uint32)
    for b in range(31, -1, -1):
        cand = thr | jnp.uint32(1 << b)
        cnt = jnp.sum((key >= cand).astype(jnp.int32), axis=1, keepdims=True)
        thr = jnp.where(cnt >= _TOP_K, cand, thr)
    tie = key == thr
    n_gt = jnp.sum((key > thr).astype(jnp.int32), axis=1, keepdims=True)
    need = _TOP_K - n_gt
    rev = (N - 1) - iota_n
    thr_r = jnp.zeros((B, 1), jnp.int32)
    for b in range(12, -1, -1):
        cand = thr_r | (1 << b)
        cnt = jnp.sum((tie & (rev >= cand)).astype(jnp.int32),
                      axis=1, keepdims=True)
        thr_r = jnp.where(cnt >= need, cand, thr_r)
    candmask = (key > thr) | (tie & (rev >= thr_r))

    cm = candmask.astype(jnp.int32)
    incl = cm
    for b in range(13):
        d = 1 << b
        incl = incl + jnp.concatenate(
            [jnp.zeros((B, d), jnp.int32), incl[:, :N - d]], axis=1)
    shift = iota_n - (incl - cm)

    arrs = [masked, iota_n, x1, y1, x2, y2, lab]
    for b in range(13):
        d = 1 << b

        def rolled(a):
            return jnp.concatenate([a[:, d:], a[:, :d]], axis=1)

        take = ((rolled(shift) >> b) & 1) == 1
        arrs = [jnp.where(take, rolled(a), a) for a in arrs]
        shift = jnp.where(take, rolled(shift) - d, shift)

    cs, cidx, cx1, cy1, cx2, cy2, clab = [a[:, :_TOP_K] for a in arrs]

    def sort_body(t, carry):
        cs, ssc, sx1, sy1, sx2, sy2, slab = carry
        mx = jnp.max(cs, axis=1, keepdims=True)
        eq = cs == mx
        mi = jnp.min(jnp.where(eq, cidx, N), axis=1, keepdims=True)
        onehot = cidx == mi

        def sel(a):
            return jnp.sum(jnp.where(onehot, a, 0.0), axis=1, keepdims=True)

        tm = lane_k == t
        ssc = jnp.where(tm, mx, ssc)
        sx1 = jnp.where(tm, sel(cx1), sx1)
        sy1 = jnp.where(tm, sel(cy1), sy1)
        sx2 = jnp.where(tm, sel(cx2), sx2)
        sy2 = jnp.where(tm, sel(cy2), sy2)
        slab = jnp.where(tm, sel(clab), slab)
        cs = jnp.where(onehot, _NEGF, cs)
        return cs, ssc, sx1, sy1, sx2, sy2, slab

    def sort_bodyu(t, carry):
        for s in range(_UNROLL):
            carry = sort_body(t * _UNROLL + s, carry)
        return carry

    z = jnp.zeros((B, _TOP_K), jnp.float32)
    carry = (cs, z, z, z, z, z, z)
    _, ssc, sx1, sy1, sx2, sy2, slab = jax.lax.fori_loop(
        0, _TOP_K // _UNROLL, sort_bodyu, carry)

    svalid = (ssc > _CONFIDENCE_THRESHOLD).astype(jnp.float32)
    area = (jnp.clip(sx2 - sx1, 0.0, None) *
            jnp.clip(sy2 - sy1, 0.0, None))

    def ext(mask, a):
        return jnp.sum(jnp.where(mask, a, 0.0), axis=1, keepdims=True)

    def nms_body(i, keep):
        oh = lane_k == i
        bx1 = ext(oh, sx1)
        by1 = ext(oh, sy1)
        bx2 = ext(oh, sx2)
        by2 = ext(oh, sy2)
        bar = ext(oh, area)
        ki = ext(oh, keep) * ext(oh, svalid)
        ltx = jnp.maximum(bx1, sx1)
        lty = jnp.maximum(by1, sy1)
        rbx = jnp.minimum(bx2, sx2)
        rby = jnp.minimum(by2, sy2)
        iw = jnp.clip(rbx - ltx, 0.0, None)
        ih = jnp.clip(rby - lty, 0.0, None)
        inter = iw * ih
        union = bar + area - inter
        iou = inter / jnp.maximum(union, 1e-9)
        sup = ((iou > _NMS_THRESHOLD) & (lane_k > i) & (ki > 0.5))
        keep = keep * (1.0 - sup.astype(jnp.float32))
        keep = jnp.where(lane_k == i, ki, keep)
        return keep

    def nms_bodyu(i, keep):
        for s in range(_UNROLL):
            keep = nms_body(i * _UNROLL + s, keep)
        return keep

    keep = jax.lax.fori_loop(0, _TOP_K // _UNROLL, nms_bodyu,
                             jnp.ones((B, _TOP_K), jnp.float32))

    def rank_body(i, rank):
        oh = lane_k == i
        yi = ext(oh, sy1)
        less = (keep > 0.5) & ((sy1 < yi) | ((sy1 == yi) & (lane_k < i)))
        ri = jnp.sum(less.astype(jnp.int32), axis=1, keepdims=True)
        return jnp.where(oh, ri, rank)

    def rank_bodyu(i, rank):
        for s in range(_UNROLL):
            rank = rank_body(i * _UNROLL + s, rank)
        return rank

    rank = jax.lax.fori_loop(0, _TOP_K // _UNROLL, rank_bodyu,
                             jnp.zeros((B, _TOP_K), jnp.int32))

    rank3 = rank.reshape(B, 1, _TOP_K)
    keep3 = keep.reshape(B, 1, _TOP_K)
    lane_o3 = jax.lax.broadcasted_iota(jnp.int32, (1, _KEEP_TOP_K, 1), 1)
    ohs = (rank3 == lane_o3) & (keep3 > 0.5)

    def scat(a):
        return jnp.sum(jnp.where(ohs, a.reshape(B, 1, _TOP_K), 0.0), axis=2)

    ox1_ref[...] = scat(sx1)
    oy1_ref[...] = scat(sy1)
    ox2_ref[...] = scat(sx2)
    oy2_ref[...] = scat(sy2)
    olab_ref[...] = scat(slab)
    osc_ref[...] = scat(ssc)


@jax.jit
def kernel(predictions, priors):
    B, N, C = predictions.shape
    pred_t = jnp.transpose(predictions, (2, 0, 1))
    pred_t = jnp.pad(pred_t, ((0, 0), (0, 0), (0, _N_PAD - N)),
                     constant_values=_NEG)
    pri_t = jnp.pad(priors.T, ((0, 0), (0, _N_PAD - N)))

    outs = pl.pallas_call(
        _detect_body,
        out_shape=[jax.ShapeDtypeStruct((B, _KEEP_TOP_K), jnp.float32)
                   for _ in range(6)],
    )(pred_t, pri_t)
    return jnp.stack(outs, axis=-1)
